# baseline (speedup 1.0000x reference)
import jax
import jax.numpy as jnp
from jax.experimental import pallas as pl
from jax.experimental.pallas import tpu as pltpu


def _abloss_body(hard_ref, soft_ref, logsum_ref, cnt_ref):
    i = pl.program_id(0)
    hard = hard_ref[...]
    mask = hard == 1
    ls = jnp.sum(jnp.where(mask, jnp.log(soft_ref[...]), 0.0))
    c = jnp.sum(hard)

    @pl.when(i == 0)
    def _init():
        logsum_ref[0, 0] = ls
        cnt_ref[0, 0] = c

    @pl.when(i != 0)
    def _acc():
        logsum_ref[0, 0] += ls
        cnt_ref[0, 0] += c


def kernel(hard_attention, soft_attention):
    B, S, D = hard_attention.shape
    rows = B * S
    R = 256
    hard2 = hard_attention.reshape(rows, D)
    soft2 = soft_attention.reshape(rows, D)
    logsum, cnt = pl.pallas_call(
        _abloss_body,
        grid=(rows // R,),
        in_specs=[
            pl.BlockSpec((R, D), lambda i: (i, 0)),
            pl.BlockSpec((R, D), lambda i: (i, 0)),
        ],
        out_specs=[
            pl.BlockSpec(memory_space=pltpu.SMEM),
            pl.BlockSpec(memory_space=pltpu.SMEM),
        ],
        out_shape=[
            jax.ShapeDtypeStruct((1, 1), jnp.float32),
            jax.ShapeDtypeStruct((1, 1), jnp.int32),
        ],
    )(hard2, soft2)
    return -logsum[0, 0] / cnt[0, 0].astype(jnp.float32)


# speedup vs baseline: 1.2015x; 1.2015x over previous
import jax
import jax.numpy as jnp
from jax.experimental import pallas as pl
from jax.experimental.pallas import tpu as pltpu


def _abloss_body(hard_ref, soft_ref, logsum_ref, cnt_ref):
    i = pl.program_id(0)
    hard = hard_ref[...]
    mask = hard == 1
    ls = jnp.sum(jnp.where(mask, jnp.log(soft_ref[...]), 0.0))
    c = jnp.sum(hard)

    @pl.when(i == 0)
    def _init():
        logsum_ref[0, 0] = ls
        cnt_ref[0, 0] = c

    @pl.when(i != 0)
    def _acc():
        logsum_ref[0, 0] += ls
        cnt_ref[0, 0] += c


def kernel(hard_attention, soft_attention):
    B, S, D = hard_attention.shape
    rows = B * S
    R = 512
    hard2 = hard_attention.reshape(rows, D)
    soft2 = soft_attention.reshape(rows, D)
    logsum, cnt = pl.pallas_call(
        _abloss_body,
        grid=(rows // R,),
        in_specs=[
            pl.BlockSpec((R, D), lambda i: (i, 0)),
            pl.BlockSpec((R, D), lambda i: (i, 0)),
        ],
        out_specs=[
            pl.BlockSpec(memory_space=pltpu.SMEM),
            pl.BlockSpec(memory_space=pltpu.SMEM),
        ],
        out_shape=[
            jax.ShapeDtypeStruct((1, 1), jnp.float32),
            jax.ShapeDtypeStruct((1, 1), jnp.int32),
        ],
    )(hard2, soft2)
    return -logsum[0, 0] / cnt[0, 0].astype(jnp.float32)
